# baseline (speedup 1.0000x reference)
import jax
import jax.numpy as jnp
from jax import lax
from jax.experimental import pallas as pl
from jax.experimental.pallas import tpu as pltpu
from jax.experimental.pallas import tpu_sc as plsc

NUM_FIXED = 2
D = 64
BATCH = 16384
SEQ = 26
NC, NS, L = 2, 16, 16
NW = NC * NS
B_PER_W = BATCH // NW
SB = 32
N_SUP = B_PER_W // SB
ROWS = SB * SEQ
NGRP = ROWS // L


def _body(idx_hbm, freeze_hbm, train_hbm, out_hbm,
          idx_v, idxc_v, rows_v, freeze_v, gsem):
    wid = lax.axis_index("s") * NC + lax.axis_index("c")
    pltpu.sync_copy(freeze_hbm, freeze_v)

    def superchunk(s, carry):
        b0 = wid * B_PER_W + s * SB
        pltpu.sync_copy(idx_hbm.at[pl.ds(b0, SB)], idx_v)

        def prep(g, c):
            p = g * L + lax.iota(jnp.int32, L)
            r = p // SEQ
            col = p % SEQ
            iv = plsc.load_gather(idx_v, [r, col])
            plsc.store_scatter(idxc_v, [r, col],
                               jnp.maximum(iv - NUM_FIXED, 0))
            return c

        lax.fori_loop(0, NGRP, prep, 0)

        for h in range(SB // 16):
            cps = [
                pltpu.async_copy(
                    train_hbm.at[idxc_v.at[h * 16 + bb]],
                    rows_v.at[h * 16 + bb],
                    gsem,
                )
                for bb in range(16)
            ]
            for cp in cps:
                cp.wait()

        def fix(g, c):
            p = g * L + lax.iota(jnp.int32, L)
            r = p // SEQ
            col = p % SEQ
            iv = plsc.load_gather(idx_v, [r, col])
            m = iv < NUM_FIXED

            @pl.when(plsc.all_reduce_population_count(m)[0] > 0)
            def _():
                ivc = jnp.minimum(iv, NUM_FIXED - 1)
                for cc in range(D):
                    cvec = jnp.full((L,), cc, jnp.int32)
                    v = plsc.load_gather(freeze_v, [ivc, cvec])
                    plsc.store_scatter(rows_v, [r, col, cvec], v, mask=m)

            return c

        lax.fori_loop(0, NGRP, fix, 0)

        pltpu.sync_copy(rows_v, out_hbm.at[pl.ds(b0, SB)])
        return carry

    lax.fori_loop(0, N_SUP, superchunk, 0)


@jax.jit
def _gather(idx, weights_freeze, weights_train):
    mesh = plsc.VectorSubcoreMesh(core_axis_name="c", subcore_axis_name="s")
    f = pl.kernel(
        _body,
        out_type=jax.ShapeDtypeStruct((BATCH, SEQ, D), jnp.float32),
        mesh=mesh,
        scratch_types=[
            pltpu.VMEM((SB, SEQ), jnp.int32),
            pltpu.VMEM((SB, SEQ), jnp.int32),
            pltpu.VMEM((SB, SEQ, D), jnp.float32),
            pltpu.VMEM((NUM_FIXED, D), jnp.float32),
            pltpu.SemaphoreType.DMA,
        ],
        compiler_params=pltpu.CompilerParams(
            needs_layout_passes=False, use_tc_tiling_on_sc=False),
    )
    return f(idx, weights_freeze, weights_train)


def kernel(idx, weights_freeze, weights_train):
    return _gather(idx.astype(jnp.int32), weights_freeze.astype(jnp.float32),
                   weights_train.astype(jnp.float32))
